# baseline (speedup 1.0000x reference)
import jax
import jax.numpy as jnp
from jax import lax
from jax.experimental import pallas as pl
from jax.experimental.pallas import tpu as pltpu
from jax.experimental.pallas import tpu_sc as plsc

_EMBED = 64
_BATCH = 16384
_C = 5
_NC = 2
_NS = 16
_NW = _NC * _NS
_BW = _BATCH // _NW
_CB = 128
_NCH = _BW // _CB
_L = 16


def _sc_body(target_hbm, context_hbm, ttab_hbm, ctab_hbm, out_hbm,
             tidx_v, cidx_v, trows_v, crows_v, out_v, sem):
    cid = lax.axis_index("c")
    sid = lax.axis_index("s")
    wid = sid * _NC + cid
    lane = lax.iota(jnp.int32, _L)

    for ch in range(_NCH):
        b0 = wid * _BW + ch * _CB
        pltpu.sync_copy(target_hbm.at[pl.ds(b0, _CB)], tidx_v)
        pltpu.sync_copy(context_hbm.at[pl.ds(b0 * _C, _CB * _C)], cidx_v)
        cps = [pltpu.async_copy(ttab_hbm.at[tidx_v], trows_v, sem)]
        for j in range(_C):
            cps.append(pltpu.async_copy(
                ctab_hbm.at[cidx_v.at[pl.ds(j * _CB, _CB)]],
                crows_v.at[pl.ds(j * _CB, _CB)], sem))
        for cp in cps:
            cp.wait()

        for g in range(_CB // _L):
            bl = g * _L + lane

            def body(e, accs, bl=bl):
                e16 = jnp.full((_L,), e, jnp.int32)
                tg = plsc.load_gather(trows_v, [bl, e16])
                return tuple(
                    accs[c] + tg * plsc.load_gather(crows_v, [bl * _C + c, e16])
                    for c in range(_C))

            accs = lax.fori_loop(
                0, _EMBED, body,
                tuple(jnp.zeros((_L,), jnp.float32) for _ in range(_C)),
                unroll=4)
            for c in range(_C):
                plsc.store_scatter(out_v, [bl * _C + c], accs[c])

        pltpu.sync_copy(out_v, out_hbm.at[pl.ds(b0 * _C, _CB * _C)])


@jax.jit
def _event2vec(target, context, target_table, context_table):
    mesh = plsc.VectorSubcoreMesh(core_axis_name="c", subcore_axis_name="s")
    run = pl.kernel(
        _sc_body,
        out_type=jax.ShapeDtypeStruct((_BATCH * _C,), jnp.float32),
        mesh=mesh,
        compiler_params=pltpu.CompilerParams(
            needs_layout_passes=False, use_tc_tiling_on_sc=False),
        scratch_types=[
            pltpu.VMEM((_CB,), jnp.int32),
            pltpu.VMEM((_CB * _C,), jnp.int32),
            pltpu.VMEM((_CB, _EMBED), jnp.float32),
            pltpu.VMEM((_CB * _C, _EMBED), jnp.float32),
            pltpu.VMEM((_CB * _C,), jnp.float32),
            pltpu.SemaphoreType.DMA,
        ],
    )
    flat = run(target, context, target_table, context_table)
    return flat.reshape(_BATCH, _C)


def kernel(target, context, target_table, context_table):
    if target.ndim == 2:
        target = jnp.squeeze(target, axis=1)
    target = target.astype(jnp.int32)
    context = context.astype(jnp.int32).reshape(-1)
    return _event2vec(target, context, target_table, context_table)
